# baseline (speedup 1.0000x reference)
import functools

import jax
import jax.numpy as jnp
from jax import lax
from jax.experimental import pallas as pl
from jax.experimental.pallas import tpu as pltpu
from jax.experimental.pallas import tpu_sc as plsc

_V = 1000
_VP = 1024
_N = 51200
_NH = _N // 2
_NC, _NS = 2, 16
_NW = _NC * _NS
_BPW = _NH // _NW
_CH = 40
_NCH = _BPW // _CH

_AUX_CHUNKS = [(k * 128, 128) for k in range(_BPW // 128)]
if _BPW % 128:
    _AUX_CHUNKS.append((_BPW - _BPW % 128, _BPW % 128))


def _row_lse_body(tab_ref, out_ref):
    x = tab_ref[...]
    m = jnp.max(x, axis=1)
    s = jnp.sum(jnp.exp(x - m[:, None]), axis=1)
    out_ref[...] = m + jnp.log(s)


_row_lse = pl.pallas_call(
    _row_lse_body,
    out_shape=jax.ShapeDtypeStruct((_V,), jnp.float32),
)


def _finalize_body(p0_ref, p1_ref, o_ref):
    s = jnp.sum(p0_ref[...], axis=(0, 1), keepdims=True)
    s = s + jnp.sum(p1_ref[...], axis=(0, 1), keepdims=True)
    o_ref[...] = s * (1.0 / _N)


_finalize = pl.pallas_call(
    _finalize_body,
    out_shape=jax.ShapeDtypeStruct((1, 1), jnp.float32),
)


def _worker_id():
    return lax.axis_index("s") * _NC + lax.axis_index("c")


def _sc_body(table_hbm, tabflat_hbm, idx_hbm, tgt_hbm, lse_hbm,
             out_hbm, part_hbm,
             idx_v, fi_v, picked_v, lseg_v, rows0, rows1, accbuf,
             sem0, sem1, sema):
    wid = _worker_id()
    base = wid * _BPW
    pltpu.sync_copy(idx_hbm.at[pl.ds(base, _BPW)], idx_v)
    pltpu.sync_copy(tgt_hbm.at[pl.ds(base, _BPW)], fi_v)

    def fi_body(i, c):
        ds = pl.ds(i * 16, 16)
        fi_v[ds] = idx_v[ds] * _V + fi_v[ds]
        return c

    lax.fori_loop(0, _BPW // 16, fi_body, jnp.int32(0))

    aux = []
    for off, ln in _AUX_CHUNKS:
        aux.append(pltpu.make_async_copy(
            tabflat_hbm.at[fi_v.at[pl.ds(off, ln)]],
            picked_v.at[pl.ds(off, ln)], sema))
        aux.append(pltpu.make_async_copy(
            lse_hbm.at[idx_v.at[pl.ds(off, ln)]],
            lseg_v.at[pl.ds(off, ln)], sema))
    for a in aux:
        a.start()

    def gather_start(c, buf, sem):
        pltpu.make_async_copy(
            table_hbm.at[idx_v.at[pl.ds(c * _CH, _CH)]], buf, sem).start()

    def gather_wait(c, buf, sem):
        pltpu.make_async_copy(
            table_hbm.at[idx_v.at[pl.ds(c * _CH, _CH)]], buf, sem).wait()

    def consume(c, buf):
        pltpu.sync_copy(buf, out_hbm.at[pl.ds(base + c * _CH, _CH)])

    gather_start(0, rows0, sem0)

    def body(i, carry):
        g = i * 2
        gather_start(g + 1, rows1, sem1)
        gather_wait(g, rows0, sem0)
        consume(g, rows0)

        @pl.when(g + 2 < _NCH)
        def _():
            gather_start(g + 2, rows0, sem0)

        gather_wait(g + 1, rows1, sem1)
        consume(g + 1, rows1)
        return carry

    lax.fori_loop(0, _NCH // 2, body, jnp.int32(0))

    for a in aux:
        a.wait()

    def acc_body(i, acc):
        ds = pl.ds(i * 16, 16)
        return acc + (lseg_v[ds] - picked_v[ds])

    acc = lax.fori_loop(0, _BPW // 16, acc_body, jnp.zeros((16,), jnp.float32))
    for k in range(8):
        accbuf[pl.ds(k * 16, 16)] = jnp.zeros((16,), jnp.float32)
    accbuf[pl.ds(0, 16)] = acc
    pltpu.sync_copy(accbuf, part_hbm.at[wid])


_sc_gather_half = functools.partial(
    pl.kernel,
    out_type=(
        jax.ShapeDtypeStruct((_NH, _VP), jnp.float32),
        jax.ShapeDtypeStruct((_NW, 128), jnp.float32),
    ),
    mesh=plsc.VectorSubcoreMesh(core_axis_name="c", subcore_axis_name="s"),
    scratch_types=[
        pltpu.VMEM((_BPW,), jnp.int32),
        pltpu.VMEM((_BPW,), jnp.int32),
        pltpu.VMEM((_BPW,), jnp.float32),
        pltpu.VMEM((_BPW,), jnp.float32),
        pltpu.VMEM((_CH, _VP), jnp.float32),
        pltpu.VMEM((_CH, _VP), jnp.float32),
        pltpu.VMEM((128,), jnp.float32),
        pltpu.SemaphoreType.DMA,
        pltpu.SemaphoreType.DMA,
        pltpu.SemaphoreType.DMA,
    ],
)(_sc_body)


_AR = 512
_HBLOCKS = _NH // _AR


def _asm0_body(h_ref, o_ref):
    o_ref[...] = h_ref[:, :_V]


_assemble0 = pl.pallas_call(
    _asm0_body,
    grid=(_HBLOCKS,),
    in_specs=[pl.BlockSpec((_AR, _VP), lambda i: (i, 0))],
    out_specs=pl.BlockSpec((_AR, _V), lambda i: (i, 0)),
    out_shape=jax.ShapeDtypeStruct((_N, _V), jnp.float32),
)


def _asm1_body(prev_ref, h_ref, o_ref):
    o_ref[...] = h_ref[:, :_V]


_assemble1 = pl.pallas_call(
    _asm1_body,
    grid=(_HBLOCKS,),
    in_specs=[
        pl.BlockSpec(memory_space=pl.ANY),
        pl.BlockSpec((_AR, _VP), lambda i: (i, 0)),
    ],
    out_specs=pl.BlockSpec((_AR, _V), lambda i: (i + _HBLOCKS, 0)),
    out_shape=jax.ShapeDtypeStruct((_N, _V), jnp.float32),
    input_output_aliases={0: 0},
)


def kernel(idx, targets, table):
    idx_flat = idx.reshape(-1)
    tgt_flat = targets.reshape(-1)
    table_pad = jnp.pad(table, ((0, 0), (0, _VP - _V)))
    tab_flat = table.reshape(-1)
    lse = _row_lse(table)
    lse_pad = jnp.pad(lse, (0, 24))
    out0, part0 = _sc_gather_half(
        table_pad, tab_flat, idx_flat[:_NH], tgt_flat[:_NH], lse_pad)
    out1, part1 = _sc_gather_half(
        table_pad, tab_flat, idx_flat[_NH:], tgt_flat[_NH:], lse_pad)
    acc = _assemble0(out0)
    logits2 = _assemble1(acc, out1)
    losses = _finalize(part0, part1)[0, 0]
    return (logits2, losses)


# speedup vs baseline: 1.4173x; 1.4173x over previous
import functools

import jax
import jax.numpy as jnp
from jax import lax
from jax.experimental import pallas as pl
from jax.experimental.pallas import tpu as pltpu
from jax.experimental.pallas import tpu_sc as plsc

_V = 1000
_VP = 1024
_VMAIN = 896
_VTAIL = 128
_N = 51200
_NC, _NS = 2, 16
_NW = _NC * _NS
_BPW = _N // _NW
_CH = 40
_NCH = _BPW // _CH

_AUX_CHUNKS = [(k * 128, 128) for k in range(_BPW // 128)]
if _BPW % 128:
    _AUX_CHUNKS.append((_BPW - _BPW % 128, _BPW % 128))


def _row_lse_body(tab_ref, out_ref):
    x = tab_ref[...]
    m = jnp.max(x, axis=1)
    s = jnp.sum(jnp.exp(x - m[:, None]), axis=1)
    out_ref[...] = m + jnp.log(s)


_row_lse = pl.pallas_call(
    _row_lse_body,
    out_shape=jax.ShapeDtypeStruct((_V,), jnp.float32),
)


def _finalize_body(p_ref, o_ref):
    o_ref[...] = jnp.sum(p_ref[...], axis=(0, 1), keepdims=True) * (1.0 / _N)


_finalize = pl.pallas_call(
    _finalize_body,
    out_shape=jax.ShapeDtypeStruct((1, 1), jnp.float32),
)


def _worker_id():
    return lax.axis_index("s") * _NC + lax.axis_index("c")


def _sc_body(table_hbm, tabflat_hbm, idx_hbm, tgt_hbm, lse_hbm,
             out_hbm, outb_hbm, part_hbm,
             idx_v, fi_v, picked_v, lseg_v, rows0, rows1, accbuf,
             sem0, sem1, sema):
    wid = _worker_id()
    base = wid * _BPW
    pltpu.sync_copy(idx_hbm.at[pl.ds(base, _BPW)], idx_v)
    pltpu.sync_copy(tgt_hbm.at[pl.ds(base, _BPW)], fi_v)

    def fi_body(i, c):
        ds = pl.ds(i * 16, 16)
        fi_v[ds] = idx_v[ds] * _V + fi_v[ds]
        return c

    lax.fori_loop(0, _BPW // 16, fi_body, jnp.int32(0))

    def gather_start(c, buf, sem):
        pltpu.make_async_copy(
            table_hbm.at[idx_v.at[pl.ds(c * _CH, _CH)]], buf, sem).start()

    def gather_wait(c, buf, sem):
        pltpu.make_async_copy(
            table_hbm.at[idx_v.at[pl.ds(c * _CH, _CH)]], buf, sem).wait()

    def consume(c, buf):
        row = base + c * _CH
        pltpu.sync_copy(buf.at[:, pl.ds(0, _VMAIN)],
                        out_hbm.at[pl.ds(row, _CH), pl.ds(0, _VMAIN)])
        pltpu.sync_copy(buf.at[:, pl.ds(_VMAIN, _VTAIL)],
                        outb_hbm.at[pl.ds(row, _CH)])

    gather_start(0, rows0, sem0)

    aux = []
    for off, ln in _AUX_CHUNKS:
        aux.append(pltpu.make_async_copy(
            tabflat_hbm.at[fi_v.at[pl.ds(off, ln)]],
            picked_v.at[pl.ds(off, ln)], sema))
        aux.append(pltpu.make_async_copy(
            lse_hbm.at[idx_v.at[pl.ds(off, ln)]],
            lseg_v.at[pl.ds(off, ln)], sema))
    for a in aux:
        a.start()

    def body(i, carry):
        g = i * 2
        gather_start(g + 1, rows1, sem1)
        gather_wait(g, rows0, sem0)
        consume(g, rows0)

        @pl.when(g + 2 < _NCH)
        def _():
            gather_start(g + 2, rows0, sem0)

        gather_wait(g + 1, rows1, sem1)
        consume(g + 1, rows1)
        return carry

    lax.fori_loop(0, _NCH // 2, body, jnp.int32(0))

    for a in aux:
        a.wait()

    def acc_body(i, acc):
        ds = pl.ds(i * 16, 16)
        return acc + (lseg_v[ds] - picked_v[ds])

    acc = lax.fori_loop(0, _BPW // 16, acc_body, jnp.zeros((16,), jnp.float32))
    for k in range(8):
        accbuf[pl.ds(k * 16, 16)] = jnp.zeros((16,), jnp.float32)
    accbuf[pl.ds(0, 16)] = acc
    pltpu.sync_copy(accbuf, part_hbm.at[wid])


_sc_gather = functools.partial(
    pl.kernel,
    out_type=(
        jax.ShapeDtypeStruct((_N, _V), jnp.float32),
        jax.ShapeDtypeStruct((_N, _VTAIL), jnp.float32),
        jax.ShapeDtypeStruct((_NW, 128), jnp.float32),
    ),
    mesh=plsc.VectorSubcoreMesh(core_axis_name="c", subcore_axis_name="s"),
    scratch_types=[
        pltpu.VMEM((_BPW,), jnp.int32),
        pltpu.VMEM((_BPW,), jnp.int32),
        pltpu.VMEM((_BPW,), jnp.float32),
        pltpu.VMEM((_BPW,), jnp.float32),
        pltpu.VMEM((_CH, _VP), jnp.float32),
        pltpu.VMEM((_CH, _VP), jnp.float32),
        pltpu.VMEM((128,), jnp.float32),
        pltpu.SemaphoreType.DMA,
        pltpu.SemaphoreType.DMA,
        pltpu.SemaphoreType.DMA,
    ],
)(_sc_body)


def kernel(idx, targets, table):
    idx_flat = idx.reshape(-1)
    tgt_flat = targets.reshape(-1)
    table_pad = jnp.pad(table, ((0, 0), (0, _VP - _V)))
    lse = _row_lse(table)
    lse_pad = jnp.pad(lse, (0, 24))
    out_main, out_b, partials = _sc_gather(
        table_pad, table.reshape(-1), idx_flat, tgt_flat, lse_pad)
    logits2 = lax.dynamic_update_slice(
        out_main, out_b[:, : _V - _VMAIN], (0, _VMAIN))
    losses = _finalize(partials)[0, 0]
    return (logits2, losses)


# speedup vs baseline: 1.4888x; 1.0505x over previous
import functools

import jax
import jax.numpy as jnp
from jax import lax
from jax.experimental import pallas as pl
from jax.experimental.pallas import tpu as pltpu
from jax.experimental.pallas import tpu_sc as plsc

_V = 1000
_VP = 1024
_N = 51200
_NC, _NS = 2, 16
_NW = _NC * _NS
_BPW = _N // _NW
_CH = 40
_NCH = _BPW // _CH
_AUXF = _BPW // 128
_AUXT = _BPW - _AUXF * 128


def _row_lse_body(tab_ref, out_ref):
    x = tab_ref[...]
    m = jnp.max(x, axis=1)
    s = jnp.sum(jnp.exp(x - m[:, None]), axis=1)
    out_ref[...] = m + jnp.log(s)


_row_lse = pl.pallas_call(
    _row_lse_body,
    out_shape=jax.ShapeDtypeStruct((_V,), jnp.float32),
)


def _finalize_body(p_ref, o_ref):
    o_ref[...] = jnp.sum(p_ref[...], axis=(0, 1), keepdims=True) * (1.0 / _N)


_finalize = pl.pallas_call(
    _finalize_body,
    out_shape=jax.ShapeDtypeStruct((1, 1), jnp.float32),
)


def _worker_id():
    return lax.axis_index("s") * _NC + lax.axis_index("c")


def _sc_body(table_hbm, tabflat_hbm, idx_hbm, tgt_hbm, lse_hbm,
             out_hbm, part_hbm,
             idx_v, fi_v, picked_v, lseg_v, rows0, rows1, accbuf,
             sem0, sem1, sema):
    wid = _worker_id()
    base = wid * _BPW
    pltpu.sync_copy(idx_hbm.at[pl.ds(base, _BPW)], idx_v)
    pltpu.sync_copy(tgt_hbm.at[pl.ds(base, _BPW)], fi_v)

    def fi_body(i, c):
        ds = pl.ds(i * 16, 16)
        fi_v[ds] = idx_v[ds] * _V + fi_v[ds]
        return c

    lax.fori_loop(0, _BPW // 16, fi_body, jnp.int32(0))

    def aux_start(off, ln):
        pltpu.make_async_copy(
            tabflat_hbm.at[fi_v.at[pl.ds(off, ln)]],
            picked_v.at[pl.ds(off, ln)], sema).start()
        pltpu.make_async_copy(
            lse_hbm.at[idx_v.at[pl.ds(off, ln)]],
            lseg_v.at[pl.ds(off, ln)], sema).start()

    def gather_start(c, buf, sem):
        pltpu.make_async_copy(
            table_hbm.at[idx_v.at[pl.ds(c * _CH, _CH)]], buf, sem).start()

    def gather_wait(c, buf, sem):
        pltpu.make_async_copy(
            table_hbm.at[idx_v.at[pl.ds(c * _CH, _CH)]], buf, sem).wait()

    def consume(c, buf):
        pltpu.sync_copy(buf, out_hbm.at[pl.ds(base + c * _CH, _CH)])

    gather_start(0, rows0, sem0)

    def body(i, carry):
        g = i * 2
        gather_start(g + 1, rows1, sem1)

        @pl.when(i < _AUXF // 2)
        def _():
            off = i * 256
            aux_start(off, 128)
            aux_start(off + 128, 128)

        gather_wait(g, rows0, sem0)
        consume(g, rows0)

        @pl.when(g + 2 < _NCH)
        def _():
            gather_start(g + 2, rows0, sem0)

        gather_wait(g + 1, rows1, sem1)
        consume(g + 1, rows1)
        return carry

    lax.fori_loop(0, _NCH // 2, body, jnp.int32(0))
    aux_start(_AUXF * 128, _AUXT)

    for k in range(_AUXF):
        pltpu.make_async_copy(
            tabflat_hbm.at[fi_v.at[pl.ds(k * 128, 128)]],
            picked_v.at[pl.ds(k * 128, 128)], sema).wait()
        pltpu.make_async_copy(
            lse_hbm.at[idx_v.at[pl.ds(k * 128, 128)]],
            lseg_v.at[pl.ds(k * 128, 128)], sema).wait()
    pltpu.make_async_copy(
        tabflat_hbm.at[fi_v.at[pl.ds(_AUXF * 128, _AUXT)]],
        picked_v.at[pl.ds(_AUXF * 128, _AUXT)], sema).wait()
    pltpu.make_async_copy(
        lse_hbm.at[idx_v.at[pl.ds(_AUXF * 128, _AUXT)]],
        lseg_v.at[pl.ds(_AUXF * 128, _AUXT)], sema).wait()

    def acc_body(i, acc):
        ds = pl.ds(i * 16, 16)
        return acc + (lseg_v[ds] - picked_v[ds])

    acc = lax.fori_loop(0, _BPW // 16, acc_body, jnp.zeros((16,), jnp.float32))
    for k in range(8):
        accbuf[pl.ds(k * 16, 16)] = jnp.zeros((16,), jnp.float32)
    accbuf[pl.ds(0, 16)] = acc
    pltpu.sync_copy(accbuf, part_hbm.at[wid])


_sc_gather = functools.partial(
    pl.kernel,
    out_type=(
        jax.ShapeDtypeStruct((_N, _VP), jnp.float32),
        jax.ShapeDtypeStruct((_NW, 128), jnp.float32),
    ),
    mesh=plsc.VectorSubcoreMesh(core_axis_name="c", subcore_axis_name="s"),
    scratch_types=[
        pltpu.VMEM((_BPW,), jnp.int32),
        pltpu.VMEM((_BPW,), jnp.int32),
        pltpu.VMEM((_BPW,), jnp.float32),
        pltpu.VMEM((_BPW,), jnp.float32),
        pltpu.VMEM((_CH, _VP), jnp.float32),
        pltpu.VMEM((_CH, _VP), jnp.float32),
        pltpu.VMEM((128,), jnp.float32),
        pltpu.SemaphoreType.DMA,
        pltpu.SemaphoreType.DMA,
        pltpu.SemaphoreType.DMA,
    ],
)(_sc_body)


def kernel(idx, targets, table):
    idx_flat = idx.reshape(-1)
    tgt_flat = targets.reshape(-1)
    table_pad = jnp.pad(table, ((0, 0), (0, _VP - _V)))
    lse = _row_lse(table)
    lse_pad = jnp.pad(lse, (0, 24))
    out_pad, partials = _sc_gather(
        table_pad, table.reshape(-1), idx_flat, tgt_flat, lse_pad)
    logits2 = out_pad[:, :_V]
    losses = _finalize(partials)[0, 0]
    return (logits2, losses)


# speedup vs baseline: 1.4948x; 1.0040x over previous
import functools

import jax
import jax.numpy as jnp
from jax import lax
from jax.experimental import pallas as pl
from jax.experimental.pallas import tpu as pltpu
from jax.experimental.pallas import tpu_sc as plsc

_V = 1000
_VP = 1024
_N = 51200
_NC, _NS = 2, 16
_NW = _NC * _NS
_BPW = _N // _NW
_CH = 16
_NCH = _BPW // _CH
_AUXF = _BPW // 128
_AUXT = _BPW - _AUXF * 128


def _row_lse_body(tab_ref, out_ref):
    x = tab_ref[...]
    m = jnp.max(x, axis=1)
    s = jnp.sum(jnp.exp(x - m[:, None]), axis=1)
    out_ref[...] = m + jnp.log(s)


_row_lse = pl.pallas_call(
    _row_lse_body,
    out_shape=jax.ShapeDtypeStruct((_V,), jnp.float32),
)


def _finalize_body(p_ref, o_ref):
    o_ref[...] = jnp.sum(p_ref[...], axis=(0, 1), keepdims=True) * (1.0 / _N)


_finalize = pl.pallas_call(
    _finalize_body,
    out_shape=jax.ShapeDtypeStruct((1, 1), jnp.float32),
)


def _worker_id():
    return lax.axis_index("s") * _NC + lax.axis_index("c")


def _sc_body(table_hbm, tabflat_hbm, idx_hbm, tgt_hbm, lse_hbm,
             out_hbm, part_hbm,
             idx_v, fi_v, picked_v, lseg_v,
             rows0, rows1, rows2, rows3, accbuf,
             gs0, gs1, gs2, gs3, ss0, ss1, ss2, ss3, sema):
    rows = (rows0, rows1, rows2, rows3)
    gsem = (gs0, gs1, gs2, gs3)
    ssem = (ss0, ss1, ss2, ss3)
    wid = _worker_id()
    base = wid * _BPW
    pltpu.sync_copy(idx_hbm.at[pl.ds(base, _BPW)], idx_v)
    pltpu.sync_copy(tgt_hbm.at[pl.ds(base, _BPW)], fi_v)

    def fi_body(i, c):
        ds = pl.ds(i * 16, 16)
        fi_v[ds] = idx_v[ds] * _V + fi_v[ds]
        return c

    lax.fori_loop(0, _BPW // 16, fi_body, jnp.int32(0))

    def aux_start(off, ln):
        pltpu.make_async_copy(
            tabflat_hbm.at[fi_v.at[pl.ds(off, ln)]],
            picked_v.at[pl.ds(off, ln)], sema).start()
        pltpu.make_async_copy(
            lse_hbm.at[idx_v.at[pl.ds(off, ln)]],
            lseg_v.at[pl.ds(off, ln)], sema).start()

    def gather_start(c, buf, sem):
        pltpu.make_async_copy(
            table_hbm.at[idx_v.at[pl.ds(c * _CH, _CH)]], buf, sem).start()

    def gather_wait(c, buf, sem):
        pltpu.make_async_copy(
            table_hbm.at[idx_v.at[pl.ds(c * _CH, _CH)]], buf, sem).wait()

    def scatter_start(c, buf, sem):
        pltpu.make_async_copy(
            buf, out_hbm.at[pl.ds(base + c * _CH, _CH)], sem).start()

    def scatter_wait(c, buf, sem):
        pltpu.make_async_copy(
            buf, out_hbm.at[pl.ds(base + c * _CH, _CH)], sem).wait()

    gather_start(0, rows[0], gsem[0])
    gather_start(1, rows[1], gsem[1])
    gather_start(2, rows[2], gsem[2])

    def body(i, carry):
        for j in range(4):
            c = i * 4 + j

            if j == 0:
                @pl.when(i < _AUXF // 4)
                def _():
                    off = i * 512
                    aux_start(off, 128)
                    aux_start(off + 128, 128)
                    aux_start(off + 256, 128)
                    aux_start(off + 384, 128)

            gather_wait(c, rows[j], gsem[j])
            scatter_start(c, rows[j], ssem[j])
            jp = (j + 3) % 4

            if j == 0:
                @pl.when(i > 0)
                def _():
                    scatter_wait(c - 1, rows[jp], ssem[jp])
            else:
                scatter_wait(c - 1, rows[jp], ssem[jp])

            @pl.when(c + 3 < _NCH)
            def _():
                gather_start(c + 3, rows[jp], gsem[jp])
        return carry

    lax.fori_loop(0, _NCH // 4, body, jnp.int32(0))
    scatter_wait(_NCH - 1, rows[(_NCH - 1) % 4], ssem[(_NCH - 1) % 4])
    aux_start(_AUXF * 128, _AUXT)

    for k in range(_AUXF):
        pltpu.make_async_copy(
            tabflat_hbm.at[fi_v.at[pl.ds(k * 128, 128)]],
            picked_v.at[pl.ds(k * 128, 128)], sema).wait()
        pltpu.make_async_copy(
            lse_hbm.at[idx_v.at[pl.ds(k * 128, 128)]],
            lseg_v.at[pl.ds(k * 128, 128)], sema).wait()
    pltpu.make_async_copy(
        tabflat_hbm.at[fi_v.at[pl.ds(_AUXF * 128, _AUXT)]],
        picked_v.at[pl.ds(_AUXF * 128, _AUXT)], sema).wait()
    pltpu.make_async_copy(
        lse_hbm.at[idx_v.at[pl.ds(_AUXF * 128, _AUXT)]],
        lseg_v.at[pl.ds(_AUXF * 128, _AUXT)], sema).wait()

    def acc_body(i, acc):
        ds = pl.ds(i * 16, 16)
        return acc + (lseg_v[ds] - picked_v[ds])

    acc = lax.fori_loop(0, _BPW // 16, acc_body, jnp.zeros((16,), jnp.float32))
    for k in range(8):
        accbuf[pl.ds(k * 16, 16)] = jnp.zeros((16,), jnp.float32)
    accbuf[pl.ds(0, 16)] = acc
    pltpu.sync_copy(accbuf, part_hbm.at[wid])


_sc_gather = functools.partial(
    pl.kernel,
    out_type=(
        jax.ShapeDtypeStruct((_N, _VP), jnp.float32),
        jax.ShapeDtypeStruct((_NW, 128), jnp.float32),
    ),
    mesh=plsc.VectorSubcoreMesh(core_axis_name="c", subcore_axis_name="s"),
    scratch_types=[
        pltpu.VMEM((_BPW,), jnp.int32),
        pltpu.VMEM((_BPW,), jnp.int32),
        pltpu.VMEM((_BPW,), jnp.float32),
        pltpu.VMEM((_BPW,), jnp.float32),
        pltpu.VMEM((_CH, _VP), jnp.float32),
        pltpu.VMEM((_CH, _VP), jnp.float32),
        pltpu.VMEM((_CH, _VP), jnp.float32),
        pltpu.VMEM((_CH, _VP), jnp.float32),
        pltpu.VMEM((128,), jnp.float32),
        pltpu.SemaphoreType.DMA,
        pltpu.SemaphoreType.DMA,
        pltpu.SemaphoreType.DMA,
        pltpu.SemaphoreType.DMA,
        pltpu.SemaphoreType.DMA,
        pltpu.SemaphoreType.DMA,
        pltpu.SemaphoreType.DMA,
        pltpu.SemaphoreType.DMA,
        pltpu.SemaphoreType.DMA,
    ],
)(_sc_body)


def kernel(idx, targets, table):
    idx_flat = idx.reshape(-1)
    tgt_flat = targets.reshape(-1)
    table_pad = jnp.pad(table, ((0, 0), (0, _VP - _V)))
    lse = _row_lse(table)
    lse_pad = jnp.pad(lse, (0, 24))
    out_pad, partials = _sc_gather(
        table_pad, table.reshape(-1), idx_flat, tgt_flat, lse_pad)
    logits2 = out_pad[:, :_V]
    losses = _finalize(partials)[0, 0]
    return (logits2, losses)
